# initial kernel scaffold (unmeasured)
import jax
import jax.numpy as jnp
from jax import lax
from jax.experimental import pallas as pl
from jax.experimental.pallas import tpu as pltpu


def kernel(
    x,
):
    def body(*refs):
        pass

    out_shape = jax.ShapeDtypeStruct(..., jnp.float32)
    return pl.pallas_call(body, out_shape=out_shape)(...)


# baseline (device time: 569973 ns/iter reference)
import jax
import jax.numpy as jnp
from jax import lax
from jax.experimental import pallas as pl
from jax.experimental.pallas import tpu as pltpu

N_DEV = 4
M = 4096
N_OUT = 1024
HALF = 2048


def kernel(x):
    assert x.shape == (1, M, N_DEV * N_OUT), x.shape

    def body(x_ref, out_ref, comm_ref, stage_ref, send_sems, recv_sems,
             local_sem):
        my = lax.axis_index("i")
        left = lax.rem(my + N_DEV - 1, N_DEV)
        right = lax.rem(my + 1, N_DEV)

        barrier_sem = pltpu.get_barrier_semaphore()
        for nbr in (left, right):
            pl.semaphore_signal(
                barrier_sem, inc=1,
                device_id=(nbr,), device_id_type=pl.DeviceIdType.MESH,
            )
        pl.semaphore_wait(barrier_sem, 2)

        for h in range(2):
            rows = pl.ds(h * HALF, HALF)

            c0 = lax.rem(my + 3, N_DEV)
            cp = pltpu.make_async_copy(
                x_ref.at[0, rows, pl.ds(c0 * N_OUT, N_OUT)],
                comm_ref.at[0],
                local_sem,
            )
            cp.start()
            cp.wait()

            for s in range(N_DEV - 1):
                send_slot = s % 2
                recv_slot = (s + 1) % 2
                rdma = pltpu.make_async_remote_copy(
                    src_ref=comm_ref.at[send_slot],
                    dst_ref=comm_ref.at[recv_slot],
                    send_sem=send_sems.at[send_slot],
                    recv_sem=recv_sems.at[recv_slot],
                    device_id=(right,),
                    device_id_type=pl.DeviceIdType.MESH,
                )
                rdma.start()

                c_recv = lax.rem(my + 2 - s + N_DEV, N_DEV)
                cp = pltpu.make_async_copy(
                    x_ref.at[0, rows, pl.ds(c_recv * N_OUT, N_OUT)],
                    stage_ref,
                    local_sem,
                )
                cp.start()
                cp.wait()

                rdma.wait()

                if s < N_DEV - 2:
                    comm_ref[recv_slot] = comm_ref[recv_slot] + stage_ref[...]
                else:
                    out_ref[rows, :] = comm_ref[recv_slot] + stage_ref[...]

    return pl.pallas_call(
        body,
        out_shape=jax.ShapeDtypeStruct((M, N_OUT), jnp.float32),
        in_specs=[pl.BlockSpec(memory_space=pl.ANY)],
        out_specs=pl.BlockSpec(memory_space=pltpu.VMEM),
        scratch_shapes=[
            pltpu.VMEM((2, HALF, N_OUT), jnp.float32),
            pltpu.VMEM((HALF, N_OUT), jnp.float32),
            pltpu.SemaphoreType.DMA((2,)),
            pltpu.SemaphoreType.DMA((2,)),
            pltpu.SemaphoreType.DMA,
        ],
        compiler_params=pltpu.CompilerParams(collective_id=0),
    )(x)


# device time: 313740 ns/iter; 1.8167x vs baseline; 1.8167x over previous
import jax
import jax.numpy as jnp
from jax import lax
from jax.experimental import pallas as pl
from jax.experimental.pallas import tpu as pltpu

N_DEV = 4
M = 4096
N_OUT = 1024
HALF = 2048


def kernel(x):
    assert x.shape == (1, M, N_DEV * N_OUT), x.shape

    def body(x_ref, out_ref, cw_ref, ccw_ref, stage_ref,
             cw_send_sems, cw_recv_sems, ccw_send_sems, ccw_recv_sems,
             local_sems):
        my = lax.axis_index("i")
        left = lax.rem(my + N_DEV - 1, N_DEV)
        right = lax.rem(my + 1, N_DEV)
        top = pl.ds(0, HALF)
        bot = pl.ds(HALF, HALF)

        cw_seed = pltpu.make_async_copy(
            x_ref.at[0, top, pl.ds(lax.rem(my + 3, N_DEV) * N_OUT, N_OUT)],
            cw_ref.at[0], local_sems.at[0])
        ccw_seed = pltpu.make_async_copy(
            x_ref.at[0, bot, pl.ds(lax.rem(my + 1, N_DEV) * N_OUT, N_OUT)],
            ccw_ref.at[0], local_sems.at[1])
        out_pre = pltpu.make_async_copy(
            x_ref.at[0, :, pl.ds(my * N_OUT, N_OUT)],
            out_ref, local_sems.at[2])
        cw_seed.start()
        ccw_seed.start()
        out_pre.start()

        barrier_sem = pltpu.get_barrier_semaphore()
        for nbr in (left, right):
            pl.semaphore_signal(
                barrier_sem, inc=1,
                device_id=(nbr,), device_id_type=pl.DeviceIdType.MESH,
            )
        pl.semaphore_wait(barrier_sem, 2)
        cw_seed.wait()
        ccw_seed.wait()

        for s in range(N_DEV - 1):
            send_slot = s % 2
            recv_slot = (s + 1) % 2
            rdma_cw = pltpu.make_async_remote_copy(
                src_ref=cw_ref.at[send_slot],
                dst_ref=cw_ref.at[recv_slot],
                send_sem=cw_send_sems.at[send_slot],
                recv_sem=cw_recv_sems.at[recv_slot],
                device_id=(right,),
                device_id_type=pl.DeviceIdType.MESH,
            )
            rdma_ccw = pltpu.make_async_remote_copy(
                src_ref=ccw_ref.at[send_slot],
                dst_ref=ccw_ref.at[recv_slot],
                send_sem=ccw_send_sems.at[send_slot],
                recv_sem=ccw_recv_sems.at[recv_slot],
                device_id=(left,),
                device_id_type=pl.DeviceIdType.MESH,
            )
            rdma_cw.start()
            rdma_ccw.start()

            if s < N_DEV - 2:
                a = lax.rem(my + 2 - s + N_DEV, N_DEV)
                b = lax.rem(my + 2 + s, N_DEV)
                st_cw = pltpu.make_async_copy(
                    x_ref.at[0, top, pl.ds(a * N_OUT, N_OUT)],
                    stage_ref, local_sems.at[0])
                st_cw.start()
                st_cw.wait()
                rdma_cw.wait()
                cw_ref[recv_slot] = cw_ref[recv_slot] + stage_ref[...]
                st_ccw = pltpu.make_async_copy(
                    x_ref.at[0, bot, pl.ds(b * N_OUT, N_OUT)],
                    stage_ref, local_sems.at[1])
                st_ccw.start()
                st_ccw.wait()
                rdma_ccw.wait()
                ccw_ref[recv_slot] = ccw_ref[recv_slot] + stage_ref[...]
            else:
                out_pre.wait()
                rdma_cw.wait()
                out_ref[top, :] = out_ref[top, :] + cw_ref[recv_slot]
                rdma_ccw.wait()
                out_ref[bot, :] = out_ref[bot, :] + ccw_ref[recv_slot]

    return pl.pallas_call(
        body,
        out_shape=jax.ShapeDtypeStruct((M, N_OUT), jnp.float32),
        in_specs=[pl.BlockSpec(memory_space=pl.ANY)],
        out_specs=pl.BlockSpec(memory_space=pltpu.VMEM),
        scratch_shapes=[
            pltpu.VMEM((2, HALF, N_OUT), jnp.float32),
            pltpu.VMEM((2, HALF, N_OUT), jnp.float32),
            pltpu.VMEM((HALF, N_OUT), jnp.float32),
            pltpu.SemaphoreType.DMA((2,)),
            pltpu.SemaphoreType.DMA((2,)),
            pltpu.SemaphoreType.DMA((2,)),
            pltpu.SemaphoreType.DMA((2,)),
            pltpu.SemaphoreType.DMA((3,)),
        ],
        compiler_params=pltpu.CompilerParams(
            collective_id=0,
            vmem_limit_bytes=60 * 1024 * 1024,
        ),
    )(x)


# device time: 296075 ns/iter; 1.9251x vs baseline; 1.0597x over previous
import jax
import jax.numpy as jnp
from jax import lax
from jax.experimental import pallas as pl
from jax.experimental.pallas import tpu as pltpu

N_DEV = 4
M = 4096
N_OUT = 1024
HALF = 2048
SUB = 1024


def kernel(x):
    assert x.shape == (1, M, N_DEV * N_OUT), x.shape

    def body(x_ref, out_ref, cw_ref, ccw_ref, stage_cw, stage_ccw,
             cw_send_sems, cw_recv_sems, ccw_send_sems, ccw_recv_sems,
             local_sems):
        my = lax.axis_index("i")
        left = lax.rem(my + N_DEV - 1, N_DEV)
        right = lax.rem(my + 1, N_DEV)

        comm = (cw_ref, ccw_ref)
        stage = (stage_cw, stage_ccw)
        send_sems = (cw_send_sems, ccw_send_sems)
        recv_sems = (cw_recv_sems, ccw_recv_sems)
        tgt = (right, left)
        row0 = (0, HALF)

        def col(c):
            return pl.ds(c * N_OUT, N_OUT)

        def recv_chunk(d, s):
            return lax.rem(my + 2 + (s if d else -s) + N_DEV, N_DEV)

        def make_rdma(d, h, j):
            return pltpu.make_async_remote_copy(
                src_ref=comm[d].at[h % 2, j],
                dst_ref=comm[d].at[(h + 1) % 2, j],
                send_sem=send_sems[d].at[h % 2, j],
                recv_sem=recv_sems[d].at[(h + 1) % 2, j],
                device_id=(tgt[d],),
                device_id_type=pl.DeviceIdType.MESH,
            )

        def stage_dma(d, s):
            return pltpu.make_async_copy(
                x_ref.at[0, pl.ds(row0[d], HALF), col(recv_chunk(d, s))],
                stage[d], local_sems.at[d])

        seeds = []
        for d in range(2):
            c0 = lax.rem(my + (1 if d else 3), N_DEV)
            for j in range(2):
                seed = pltpu.make_async_copy(
                    x_ref.at[0, pl.ds(row0[d] + j * SUB, SUB), col(c0)],
                    comm[d].at[0, j], local_sems.at[2 + 2 * d + j])
                seed.start()
                seeds.append(seed)
        stages = [stage_dma(d, 0) for d in range(2)]
        for st in stages:
            st.start()

        barrier_sem = pltpu.get_barrier_semaphore()
        for nbr in (left, right):
            pl.semaphore_signal(
                barrier_sem, inc=1,
                device_id=(nbr,), device_id_type=pl.DeviceIdType.MESH,
            )
        pl.semaphore_wait(barrier_sem, 2)

        sends = {}
        for seed in seeds:
            seed.wait()
        for j in range(2):
            for d in range(2):
                r = make_rdma(d, 0, j)
                r.start()
                sends[(0, d, j)] = r

        out_dmas = []
        for s in range(N_DEV - 1):
            recv_slot = (s + 1) % 2
            for d in range(2):
                stages[d].wait()
            for j in range(2):
                for d in range(2):
                    make_rdma(d, s, j).wait_recv()
                    comm[d][recv_slot, j] = (
                        comm[d][recv_slot, j] + stage[d][pl.ds(j * SUB, SUB)]
                    )
                    if s < N_DEV - 2:
                        if s >= 1:
                            sends[(s - 1, d, j)].wait_send()
                        r = make_rdma(d, s + 1, j)
                        r.start()
                        sends[(s + 1, d, j)] = r
                    else:
                        cp = pltpu.make_async_copy(
                            comm[d].at[recv_slot, j],
                            out_ref.at[pl.ds(row0[d] + j * SUB, SUB), :],
                            local_sems.at[6 + len(out_dmas)])
                        cp.start()
                        out_dmas.append(cp)
            if s < N_DEV - 2:
                stages = [stage_dma(d, s + 1) for d in range(2)]
                for st in stages:
                    st.start()

        for j in range(2):
            for d in range(2):
                for h in (1, 2):
                    sends[(h, d, j)].wait_send()
        for cp in out_dmas:
            cp.wait()

    return pl.pallas_call(
        body,
        out_shape=jax.ShapeDtypeStruct((M, N_OUT), jnp.float32),
        in_specs=[pl.BlockSpec(memory_space=pl.ANY)],
        out_specs=pl.BlockSpec(memory_space=pl.ANY),
        scratch_shapes=[
            pltpu.VMEM((2, 2, SUB, N_OUT), jnp.float32),
            pltpu.VMEM((2, 2, SUB, N_OUT), jnp.float32),
            pltpu.VMEM((HALF, N_OUT), jnp.float32),
            pltpu.VMEM((HALF, N_OUT), jnp.float32),
            pltpu.SemaphoreType.DMA((2, 2)),
            pltpu.SemaphoreType.DMA((2, 2)),
            pltpu.SemaphoreType.DMA((2, 2)),
            pltpu.SemaphoreType.DMA((2, 2)),
            pltpu.SemaphoreType.DMA((10,)),
        ],
        compiler_params=pltpu.CompilerParams(
            collective_id=0,
            vmem_limit_bytes=60 * 1024 * 1024,
        ),
    )(x)


# device time: 294656 ns/iter; 1.9344x vs baseline; 1.0048x over previous
import jax
import jax.numpy as jnp
from jax import lax
from jax.experimental import pallas as pl
from jax.experimental.pallas import tpu as pltpu

N_DEV = 4
M = 4096
N_OUT = 1024
HALF = 2048
K = 4
SUB = HALF // K


def kernel(x):
    assert x.shape == (1, M, N_DEV * N_OUT), x.shape

    def body(x_ref, out_ref, cw_ref, ccw_ref, stage_cw, stage_ccw,
             cw_send_sems, cw_recv_sems, ccw_send_sems, ccw_recv_sems,
             local_sems):
        my = lax.axis_index("i")
        left = lax.rem(my + N_DEV - 1, N_DEV)
        right = lax.rem(my + 1, N_DEV)

        comm = (cw_ref, ccw_ref)
        stage = (stage_cw, stage_ccw)
        send_sems = (cw_send_sems, ccw_send_sems)
        recv_sems = (cw_recv_sems, ccw_recv_sems)
        tgt = (right, left)
        row0 = (0, HALF)

        def col(c):
            return pl.ds(c * N_OUT, N_OUT)

        def recv_chunk(d, s):
            return lax.rem(my + 2 + (s if d else -s) + N_DEV, N_DEV)

        def make_rdma(d, h, j):
            return pltpu.make_async_remote_copy(
                src_ref=comm[d].at[h % 2, j],
                dst_ref=comm[d].at[(h + 1) % 2, j],
                send_sem=send_sems[d].at[h % 2, j],
                recv_sem=recv_sems[d].at[(h + 1) % 2, j],
                device_id=(tgt[d],),
                device_id_type=pl.DeviceIdType.MESH,
            )

        def stage_dma(d, s):
            return pltpu.make_async_copy(
                x_ref.at[0, pl.ds(row0[d], HALF), col(recv_chunk(d, s))],
                stage[d], local_sems.at[d])

        seeds = []
        for d in range(2):
            c0 = lax.rem(my + (1 if d else 3), N_DEV)
            for j in range(K):
                seed = pltpu.make_async_copy(
                    x_ref.at[0, pl.ds(row0[d] + j * SUB, SUB), col(c0)],
                    comm[d].at[0, j], local_sems.at[2 + K * d + j])
                seed.start()
                seeds.append(seed)
        stages = [stage_dma(d, 0) for d in range(2)]
        for st in stages:
            st.start()

        barrier_sem = pltpu.get_barrier_semaphore()
        for nbr in (left, right):
            pl.semaphore_signal(
                barrier_sem, inc=1,
                device_id=(nbr,), device_id_type=pl.DeviceIdType.MESH,
            )
        pl.semaphore_wait(barrier_sem, 2)

        sends = {}
        for seed in seeds:
            seed.wait()
        for j in range(K):
            for d in range(2):
                r = make_rdma(d, 0, j)
                r.start()
                sends[(0, d, j)] = r

        out_dmas = []
        for s in range(N_DEV - 1):
            recv_slot = (s + 1) % 2
            for d in range(2):
                stages[d].wait()
            for j in range(K):
                for d in range(2):
                    make_rdma(d, s, j).wait_recv()
                    comm[d][recv_slot, j] = (
                        comm[d][recv_slot, j] + stage[d][pl.ds(j * SUB, SUB)]
                    )
                    if s < N_DEV - 2:
                        if s >= 1:
                            sends[(s - 1, d, j)].wait_send()
                        r = make_rdma(d, s + 1, j)
                        r.start()
                        sends[(s + 1, d, j)] = r
                    else:
                        cp = pltpu.make_async_copy(
                            comm[d].at[recv_slot, j],
                            out_ref.at[pl.ds(row0[d] + j * SUB, SUB), :],
                            local_sems.at[2 + 2 * K + len(out_dmas)])
                        cp.start()
                        out_dmas.append(cp)
            if s < N_DEV - 2:
                stages = [stage_dma(d, s + 1) for d in range(2)]
                for st in stages:
                    st.start()

        for j in range(K):
            for d in range(2):
                for h in (1, 2):
                    sends[(h, d, j)].wait_send()
        for cp in out_dmas:
            cp.wait()

    return pl.pallas_call(
        body,
        out_shape=jax.ShapeDtypeStruct((M, N_OUT), jnp.float32),
        in_specs=[pl.BlockSpec(memory_space=pl.ANY)],
        out_specs=pl.BlockSpec(memory_space=pl.ANY),
        scratch_shapes=[
            pltpu.VMEM((2, K, SUB, N_OUT), jnp.float32),
            pltpu.VMEM((2, K, SUB, N_OUT), jnp.float32),
            pltpu.VMEM((HALF, N_OUT), jnp.float32),
            pltpu.VMEM((HALF, N_OUT), jnp.float32),
            pltpu.SemaphoreType.DMA((2, K)),
            pltpu.SemaphoreType.DMA((2, K)),
            pltpu.SemaphoreType.DMA((2, K)),
            pltpu.SemaphoreType.DMA((2, K)),
            pltpu.SemaphoreType.DMA((2 + 4 * K,)),
        ],
        compiler_params=pltpu.CompilerParams(
            collective_id=0,
            vmem_limit_bytes=60 * 1024 * 1024,
        ),
    )(x)
